# baseline (speedup 1.0000x reference)
import functools

import jax
import jax.numpy as jnp
from jax import lax
from jax.experimental import pallas as pl
from jax.experimental.pallas import tpu as pltpu
from jax.experimental.pallas import tpu_sc as plsc

NC = 2
NS = 16
NW = NC * NS
L = 16

C = 20
H = 16
ROW = C * H

K = 64


def _pc_linear_body(x_hbm, idx_hbm, w_hbm, b_hbm, out_hbm,
                    idx_v, w_v, b_v, x_v, o_v, sem):
    rows = x_hbm.shape[0]
    rows_per_w = rows // NW
    chunks = rows_per_w // K
    wid = lax.axis_index("s") * NC + lax.axis_index("c")
    base0 = wid * rows_per_w

    pltpu.sync_copy(w_hbm, w_v)
    pltpu.sync_copy(b_hbm, b_v)

    iota = lax.iota(jnp.int32, L)
    dnums = lax.GatherDimensionNumbers(
        offset_dims=(), collapsed_slice_dims=(0,), start_index_map=(0,))

    def _bcast_lane(vec, j):
        return lax.gather(
            vec, (iota * 0 + j)[:, None], dnums, slice_sizes=(1,),
            mode=lax.GatherScatterMode.PROMISE_IN_BOUNDS)

    def chunk_body(ci, carry):
        base = base0 + ci * K
        ci_idx = pltpu.async_copy(idx_hbm.at[pl.ds(base, K)], idx_v, sem)
        ci_x = pltpu.async_copy(x_hbm.at[pl.ds(base, K)], x_v, sem)
        ci_idx.wait()
        ci_x.wait()

        def group_body(g, gcarry):
            idx_vec = idx_v[pl.ds(g * L, L)]
            for j in range(L):
                addr = _bcast_lane(idx_vec, j) * H + iota
                wr = plsc.load_gather(w_v, [addr])
                br = plsc.load_gather(b_v, [addr])
                r = g * L + j
                for c in range(C):
                    o_v[r, pl.ds(c * L, L)] = (
                        wr * x_v[r, pl.ds(c * L, L)] + br)
            return gcarry

        lax.fori_loop(0, K // L, group_body, 0, unroll=False)
        pltpu.sync_copy(o_v, out_hbm.at[pl.ds(base, K)])
        return carry

    lax.fori_loop(0, chunks, chunk_body, 0, unroll=False)


@functools.partial(jax.jit, static_argnums=())
def kernel(x, periodic_indices, w, b):
    n, t, c, h = x.shape
    rows = n * t
    x2 = x.reshape(rows, c * h)
    idx = periodic_indices.reshape(rows).astype(jnp.int32)

    mesh = plsc.VectorSubcoreMesh(core_axis_name="c", subcore_axis_name="s")
    run = pl.kernel(
        _pc_linear_body,
        out_type=jax.ShapeDtypeStruct((rows, c * h), jnp.float32),
        mesh=mesh,
        compiler_params=pltpu.CompilerParams(needs_layout_passes=False),
        scratch_types=[
            pltpu.VMEM((K,), jnp.int32),
            pltpu.VMEM((168 * H,), jnp.float32),
            pltpu.VMEM((168 * H,), jnp.float32),
            pltpu.VMEM((K, ROW), jnp.float32),
            pltpu.VMEM((K, ROW), jnp.float32),
            pltpu.SemaphoreType.DMA,
        ],
    )
    out = run(x2, idx, w.reshape(-1), b.reshape(-1))
    return out.reshape(n, t, c, h)
